# baseline (speedup 1.0000x reference)
import functools

import jax
import jax.numpy as jnp
from jax import lax
from jax.experimental import pallas as pl
from jax.experimental.pallas import tpu as pltpu
from jax.experimental.pallas import tpu_sc as plsc

NUM_EMB = 1_000_000
DIM = 32
BATCH = 16384
HIST = 50

NC = 2
NS = 16
NW = NC * NS
CB = 128
NCHUNK = BATCH // (NW * CB)
G = 5
SPC = HIST // G
STEPS = NCHUNK * SPC
N_PAIR = STEPS // 2

_mesh = plsc.VectorSubcoreMesh(core_axis_name="c", subcore_axis_name="s")


@functools.partial(
    pl.kernel,
    mesh=_mesh,
    out_type=jax.ShapeDtypeStruct((HIST, DIM // 8, BATCH // CB, 8, CB),
                                  jnp.float32),
    scratch_types=[
        pltpu.VMEM((CB, HIST), jnp.int32),
        pltpu.VMEM((NCHUNK, HIST, CB), jnp.int32),
        pltpu.VMEM((2, G, CB, DIM), jnp.float32),
        pltpu.VMEM((2, G, DIM // 8, 8, CB), jnp.float32),
        pltpu.SemaphoreType.DMA,
        pltpu.SemaphoreType.DMA,
        pltpu.SemaphoreType.DMA,
        pltpu.SemaphoreType.DMA,
        pltpu.SemaphoreType.DMA,
    ],
    compiler_params=pltpu.CompilerParams(use_tc_tiling_on_sc=False,
                                         needs_layout_passes=False),
)
def _gather_kernel(lab_hbm, w_hbm, out_hbm, lab_v, idxT_v, rows_v, outT_v,
                   sem_i, sem_g0, sem_g1, sem_o0, sem_o1):
    wid = lax.axis_index("s") * NC + lax.axis_index("c")
    sem_g = (sem_g0, sem_g1)
    sem_o = (sem_o0, sem_o1)
    iota = lax.iota(jnp.int32, 16)
    zeros = jnp.zeros((16,), jnp.int32)

    for c in range(NCHUNK):
        bt = wid * NCHUNK + c
        pltpu.async_copy(lab_hbm.at[pl.ds(bt * CB, CB)], lab_v, sem_i).wait()

        def lab_t(i, carry, c=c):
            h = i // (CB // 16)
            b0 = (i % (CB // 16)) * 16
            vals = plsc.load_gather(lab_v, [b0 + iota, zeros + h])
            idxT_v[c, h, pl.ds(b0, 16)] = vals
            return carry

        lax.fori_loop(0, HIST * (CB // 16), lab_t, 0)

    def fire_gathers(s, u):
        c = s // SPC
        h0 = (s % SPC) * G
        for hg in range(G):
            pltpu.async_copy(w_hbm.at[idxT_v.at[c, h0 + hg]],
                             rows_v.at[u, hg], sem_g[u])

    def wait_gathers(s, u):
        c = s // SPC
        h0 = (s % SPC) * G
        for hg in range(G):
            pltpu.make_async_copy(w_hbm.at[idxT_v.at[c, h0 + hg]],
                                  rows_v.at[u, hg], sem_g[u]).wait()

    def transpose_rows(u):
        def tb(i, carry):
            b0 = i * 16
            bidx = b0 + iota
            for hg in range(G):
                for dt in range(DIM // 8):
                    for di in range(8):
                        vals = plsc.load_gather(
                            rows_v.at[u],
                            [zeros + hg, bidx, zeros + (dt * 8 + di)])
                        outT_v[u, hg, dt, di, pl.ds(b0, 16)] = vals
            return carry

        lax.fori_loop(0, CB // 16, tb, 0)

    def out_dst(s):
        c = s // SPC
        h0 = (s % SPC) * G
        bt = wid * NCHUNK + c
        return out_hbm.at[pl.ds(h0, G), :, bt]

    def fire_out(s, u):
        pltpu.async_copy(outT_v.at[u], out_dst(s), sem_o[u])

    def wait_out(s, u):
        pltpu.make_async_copy(outT_v.at[u], out_dst(s), sem_o[u]).wait()

    fire_gathers(0, 0)

    def body(p, carry):
        for u in (0, 1):
            v = 1 - u
            s = 2 * p + u
            wait_gathers(s, u)

            @pl.when(s + 1 < STEPS)
            def _():
                fire_gathers(s + 1, v)

            @pl.when(s >= 1)
            def _():
                wait_out(s - 1, v)

            transpose_rows(u)
            fire_out(s, u)
        return carry

    lax.fori_loop(0, N_PAIR, body, 0)
    wait_out(STEPS - 1, 1)


def kernel(labels, weight):
    o = _gather_kernel(labels.astype(jnp.int32), weight)
    return o.transpose(2, 4, 0, 1, 3).reshape(BATCH, HIST, DIM)


# speedup vs baseline: 1.0232x; 1.0232x over previous
import functools

import jax
import jax.numpy as jnp
from jax import lax
from jax.experimental import pallas as pl
from jax.experimental.pallas import tpu as pltpu
from jax.experimental.pallas import tpu_sc as plsc

NUM_EMB = 1_000_000
DIM = 32
BATCH = 16384
HIST = 50

NC = 2
NS = 16
NW = NC * NS
CB = 128
NCHUNK = BATCH // (NW * CB)
G = 5
SPC = HIST // G
STEPS = NCHUNK * SPC
N_PAIR = STEPS // 2

_mesh = plsc.VectorSubcoreMesh(core_axis_name="c", subcore_axis_name="s")


@functools.partial(
    pl.kernel,
    mesh=_mesh,
    out_type=jax.ShapeDtypeStruct((HIST, DIM // 8, BATCH // CB, 8, CB),
                                  jnp.float32),
    scratch_types=[
        pltpu.VMEM((CB, HIST), jnp.int32),
        pltpu.VMEM((NCHUNK, HIST, CB), jnp.int32),
        pltpu.VMEM((2 * G * CB, DIM), jnp.float32),
        pltpu.VMEM((2, G, DIM // 8, 8, CB), jnp.float32),
        pltpu.SemaphoreType.DMA,
        pltpu.SemaphoreType.DMA,
        pltpu.SemaphoreType.DMA,
        pltpu.SemaphoreType.DMA,
        pltpu.SemaphoreType.DMA,
    ],
    compiler_params=pltpu.CompilerParams(use_tc_tiling_on_sc=False,
                                         needs_layout_passes=False,
                                         disable_bounds_checks=True),
)
def _gather_kernel(lab_hbm, w_hbm, out_hbm, lab_v, idxT_v, rows_v, outT_v,
                   sem_i, sem_g0, sem_g1, sem_o0, sem_o1):
    wid = lax.axis_index("s") * NC + lax.axis_index("c")
    sem_g = (sem_g0, sem_g1)
    sem_o = (sem_o0, sem_o1)
    iota = lax.iota(jnp.int32, 16)
    zeros = jnp.zeros((16,), jnp.int32)

    for c in range(NCHUNK):
        bt = wid * NCHUNK + c
        pltpu.async_copy(lab_hbm.at[pl.ds(bt * CB, CB)], lab_v, sem_i).wait()

        def lab_t(i, carry, c=c):
            h = i // (CB // 16)
            b0 = (i % (CB // 16)) * 16
            vals = plsc.load_gather(lab_v, [b0 + iota, zeros + h])
            idxT_v[c, h, pl.ds(b0, 16)] = vals
            return carry

        lax.fori_loop(0, HIST * (CB // 16), lab_t, 0)

    def fire_gathers(s, u):
        c = s // SPC
        h0 = (s % SPC) * G
        for hg in range(G):
            pltpu.async_copy(w_hbm.at[idxT_v.at[c, h0 + hg]],
                             rows_v.at[pl.ds((u * G + hg) * CB, CB)],
                             sem_g[u])

    def wait_gathers(s, u):
        c = s // SPC
        h0 = (s % SPC) * G
        for hg in range(G):
            pltpu.make_async_copy(w_hbm.at[idxT_v.at[c, h0 + hg]],
                                  rows_v.at[pl.ds((u * G + hg) * CB, CB)],
                                  sem_g[u]).wait()

    dcols = [zeros + d for d in range(DIM)]

    def transpose_rows(u):
        def tb(i, carry):
            b0 = i * 16
            bidx = b0 + iota
            for hg in range(G):
                rvec = bidx + (u * G + hg) * CB
                for dt in range(DIM // 8):
                    for di in range(8):
                        vals = plsc.load_gather(rows_v,
                                                [rvec, dcols[dt * 8 + di]])
                        outT_v[u, hg, dt, di, pl.ds(b0, 16)] = vals
            return carry

        lax.fori_loop(0, CB // 16, tb, 0)

    def out_dst(s):
        c = s // SPC
        h0 = (s % SPC) * G
        bt = wid * NCHUNK + c
        return out_hbm.at[pl.ds(h0, G), :, bt]

    def fire_out(s, u):
        pltpu.async_copy(outT_v.at[u], out_dst(s), sem_o[u])

    def wait_out(s, u):
        pltpu.make_async_copy(outT_v.at[u], out_dst(s), sem_o[u]).wait()

    fire_gathers(0, 0)

    def body(p, carry):
        for u in (0, 1):
            v = 1 - u
            s = 2 * p + u
            wait_gathers(s, u)

            @pl.when(s + 1 < STEPS)
            def _():
                fire_gathers(s + 1, v)

            transpose_rows(u)
            fire_out(s, u)

            @pl.when(s >= 1)
            def _():
                wait_out(s - 1, v)
        return carry

    lax.fori_loop(0, N_PAIR, body, 0)
    wait_out(STEPS - 1, 1)


def kernel(labels, weight):
    o = _gather_kernel(labels.astype(jnp.int32), weight)
    return o.transpose(2, 4, 0, 1, 3).reshape(BATCH, HIST, DIM)


# speedup vs baseline: 1.0706x; 1.0463x over previous
import functools

import jax
import jax.numpy as jnp
from jax import lax
from jax.experimental import pallas as pl
from jax.experimental.pallas import tpu as pltpu
from jax.experimental.pallas import tpu_sc as plsc

NUM_EMB = 1_000_000
DIM = 32
BATCH = 16384
HIST = 50

NC = 2
NS = 16
NW = NC * NS
CB = 128
NCHUNK = BATCH // (NW * CB)
G = 5
SPC = HIST // G
STEPS = NCHUNK * SPC
N_PAIR = STEPS // 2

_mesh = plsc.VectorSubcoreMesh(core_axis_name="c", subcore_axis_name="s")


@functools.partial(
    pl.kernel,
    mesh=_mesh,
    out_type=jax.ShapeDtypeStruct((HIST, DIM // 8, BATCH // CB, 8, CB),
                                  jnp.float32),
    scratch_types=[
        pltpu.VMEM((CB, HIST), jnp.int32),
        pltpu.VMEM((NCHUNK, HIST, CB), jnp.int32),
        pltpu.VMEM((2 * G * CB, DIM), jnp.float32),
        pltpu.VMEM((2, G, DIM // 8, 8, CB), jnp.float32),
        pltpu.SemaphoreType.DMA,
        pltpu.SemaphoreType.DMA,
        pltpu.SemaphoreType.DMA,
        pltpu.SemaphoreType.DMA,
        pltpu.SemaphoreType.DMA,
    ],
    compiler_params=pltpu.CompilerParams(use_tc_tiling_on_sc=False,
                                         needs_layout_passes=False,
                                         disable_bounds_checks=True),
)
def _gather_kernel(lab_hbm, w_hbm, out_hbm, lab_v, idxT_v, rows_v, outT_v,
                   sem_i, sem_g0, sem_g1, sem_o0, sem_o1):
    wid = lax.axis_index("s") * NC + lax.axis_index("c")
    sem_g = (sem_g0, sem_g1)
    sem_o = (sem_o0, sem_o1)
    iota = lax.iota(jnp.int32, 16)
    zeros = jnp.zeros((16,), jnp.int32)

    for c in range(NCHUNK):
        bt = wid * NCHUNK + c
        pltpu.async_copy(lab_hbm.at[pl.ds(bt * CB, CB)], lab_v, sem_i).wait()

        def lab_t(i, carry, c=c):
            h = i // (CB // 16)
            b0 = (i % (CB // 16)) * 16
            vals = plsc.load_gather(lab_v, [b0 + iota, zeros + h])
            idxT_v[c, h, pl.ds(b0, 16)] = vals
            return carry

        lax.fori_loop(0, HIST * (CB // 16), lab_t, 0)

    def fire_gathers(s, u):
        c = s // SPC
        h0 = (s % SPC) * G
        for hg in range(G):
            pltpu.async_copy(w_hbm.at[idxT_v.at[c, h0 + hg]],
                             rows_v.at[pl.ds((u * G + hg) * CB, CB)],
                             sem_g[u])

    def wait_gathers(s, u):
        c = s // SPC
        h0 = (s % SPC) * G
        for hg in range(G):
            pltpu.make_async_copy(w_hbm.at[idxT_v.at[c, h0 + hg]],
                                  rows_v.at[pl.ds((u * G + hg) * CB, CB)],
                                  sem_g[u]).wait()

    dcols = [zeros + d for d in range(DIM)]

    def transpose_rows(u):
        @plsc.parallel_loop(0, CB // 16, unroll=2)
        def tb(i):
            b0 = i * 16
            bidx = b0 + iota
            for hg in range(G):
                rvec = bidx + (u * G + hg) * CB
                for dt in range(DIM // 8):
                    for di in range(8):
                        vals = plsc.load_gather(rows_v,
                                                [rvec, dcols[dt * 8 + di]])
                        outT_v[u, hg, dt, di, pl.ds(b0, 16)] = vals

    def out_dst(s):
        c = s // SPC
        h0 = (s % SPC) * G
        bt = wid * NCHUNK + c
        return out_hbm.at[pl.ds(h0, G), :, bt]

    def fire_out(s, u):
        pltpu.async_copy(outT_v.at[u], out_dst(s), sem_o[u])

    def wait_out(s, u):
        pltpu.make_async_copy(outT_v.at[u], out_dst(s), sem_o[u]).wait()

    fire_gathers(0, 0)

    def body(p, carry):
        for u in (0, 1):
            v = 1 - u
            s = 2 * p + u
            wait_gathers(s, u)

            @pl.when(s + 1 < STEPS)
            def _():
                fire_gathers(s + 1, v)

            transpose_rows(u)
            fire_out(s, u)

            @pl.when(s >= 1)
            def _():
                wait_out(s - 1, v)
        return carry

    lax.fori_loop(0, N_PAIR, body, 0)
    wait_out(STEPS - 1, 1)


def kernel(labels, weight):
    o = _gather_kernel(labels.astype(jnp.int32), weight)
    return o.transpose(2, 4, 0, 1, 3).reshape(BATCH, HIST, DIM)


# speedup vs baseline: 1.0904x; 1.0184x over previous
import functools

import jax
import jax.numpy as jnp
from jax import lax
from jax.experimental import pallas as pl
from jax.experimental.pallas import tpu as pltpu
from jax.experimental.pallas import tpu_sc as plsc

NUM_EMB = 1_000_000
DIM = 32
BATCH = 16384
HIST = 50

NC = 2
NS = 16
NW = NC * NS
CB = 128
NCHUNK = BATCH // (NW * CB)
G = 5
SPC = HIST // G
STEPS = NCHUNK * SPC
N_PAIR = STEPS // 2

_mesh = plsc.VectorSubcoreMesh(core_axis_name="c", subcore_axis_name="s")


@functools.partial(
    pl.kernel,
    mesh=_mesh,
    out_type=jax.ShapeDtypeStruct((HIST, DIM // 8, BATCH // CB, 8, CB),
                                  jnp.float32),
    scratch_types=[
        pltpu.VMEM((CB, HIST), jnp.int32),
        pltpu.VMEM((NCHUNK, HIST, CB), jnp.int32),
        pltpu.VMEM((2 * G * CB, DIM), jnp.float32),
        pltpu.VMEM((2, G, DIM // 8, 8, CB), jnp.float32),
        pltpu.SemaphoreType.DMA,
        pltpu.SemaphoreType.DMA,
        pltpu.SemaphoreType.DMA,
        pltpu.SemaphoreType.DMA,
        pltpu.SemaphoreType.DMA,
    ],
    compiler_params=pltpu.CompilerParams(use_tc_tiling_on_sc=False,
                                         needs_layout_passes=False,
                                         disable_bounds_checks=True),
)
def _gather_kernel(lab_hbm, w_hbm, out_hbm, lab_v, idxT_v, rows_v, outT_v,
                   sem_i, sem_g0, sem_g1, sem_o0, sem_o1):
    wid = lax.axis_index("s") * NC + lax.axis_index("c")
    sem_g = (sem_g0, sem_g1)
    sem_o = (sem_o0, sem_o1)
    iota = lax.iota(jnp.int32, 16)
    zeros = jnp.zeros((16,), jnp.int32)

    for c in range(NCHUNK):
        bt = wid * NCHUNK + c
        pltpu.async_copy(lab_hbm.at[pl.ds(bt * CB, CB)], lab_v, sem_i).wait()

        def lab_t(i, carry, c=c):
            h = i // (CB // 16)
            b0 = (i % (CB // 16)) * 16
            vals = plsc.load_gather(lab_v, [b0 + iota, zeros + h])
            idxT_v[c, h, pl.ds(b0, 16)] = vals
            return carry

        lax.fori_loop(0, HIST * (CB // 16), lab_t, 0)

    def fire_gathers(s, u):
        c = s // SPC
        h0 = (s % SPC) * G
        for hg in range(G):
            pltpu.async_copy(w_hbm.at[idxT_v.at[c, h0 + hg]],
                             rows_v.at[pl.ds((u * G + hg) * CB, CB)],
                             sem_g[u])

    def wait_gathers(s, u):
        c = s // SPC
        h0 = (s % SPC) * G
        for hg in range(G):
            pltpu.make_async_copy(w_hbm.at[idxT_v.at[c, h0 + hg]],
                                  rows_v.at[pl.ds((u * G + hg) * CB, CB)],
                                  sem_g[u]).wait()

    dcols = [zeros + d for d in range(DIM)]

    def transpose_rows(u):
        @plsc.parallel_loop(0, CB // 16, unroll=4)
        def tb(i):
            b0 = i * 16
            bidx = b0 + iota
            for hg in range(G):
                rvec = bidx + (u * G + hg) * CB
                for dt in range(DIM // 8):
                    for di in range(8):
                        vals = plsc.load_gather(rows_v,
                                                [rvec, dcols[dt * 8 + di]])
                        outT_v[u, hg, dt, di, pl.ds(b0, 16)] = vals

    def out_dst(s):
        c = s // SPC
        h0 = (s % SPC) * G
        bt = wid * NCHUNK + c
        return out_hbm.at[pl.ds(h0, G), :, bt]

    def fire_out(s, u):
        pltpu.async_copy(outT_v.at[u], out_dst(s), sem_o[u])

    def wait_out(s, u):
        pltpu.make_async_copy(outT_v.at[u], out_dst(s), sem_o[u]).wait()

    fire_gathers(0, 0)

    def body(p, carry):
        for u in (0, 1):
            v = 1 - u
            s = 2 * p + u
            wait_gathers(s, u)

            @pl.when(s + 1 < STEPS)
            def _():
                fire_gathers(s + 1, v)

            transpose_rows(u)
            fire_out(s, u)

            @pl.when(s >= 1)
            def _():
                wait_out(s - 1, v)
        return carry

    lax.fori_loop(0, N_PAIR, body, 0)
    wait_out(STEPS - 1, 1)


def kernel(labels, weight):
    o = _gather_kernel(labels.astype(jnp.int32), weight)
    return o.transpose(2, 4, 0, 1, 3).reshape(BATCH, HIST, DIM)


# speedup vs baseline: 1.1359x; 1.0418x over previous
import functools

import jax
import jax.numpy as jnp
from jax import lax
from jax.experimental import pallas as pl
from jax.experimental.pallas import tpu as pltpu
from jax.experimental.pallas import tpu_sc as plsc

NUM_EMB = 1_000_000
DIM = 32
BATCH = 16384
HIST = 50

NC = 2
NS = 16
NW = NC * NS
CB = 128
NCHUNK = BATCH // (NW * CB)
G = 5
SPC = HIST // G
STEPS = NCHUNK * SPC
N_PAIR = STEPS // 2

_mesh = plsc.VectorSubcoreMesh(core_axis_name="c", subcore_axis_name="s")


@functools.partial(
    pl.kernel,
    mesh=_mesh,
    out_type=jax.ShapeDtypeStruct((HIST, DIM // 8, BATCH // CB, 8, CB),
                                  jnp.float32),
    scratch_types=[
        pltpu.VMEM((CB, HIST), jnp.int32),
        pltpu.VMEM((NCHUNK, HIST, CB), jnp.int32),
        pltpu.VMEM((2 * G * CB, DIM), jnp.float32),
        pltpu.VMEM((2, G, DIM // 8, 8, CB), jnp.float32),
        pltpu.SemaphoreType.DMA,
        pltpu.SemaphoreType.DMA,
        pltpu.SemaphoreType.DMA,
        pltpu.SemaphoreType.DMA,
        pltpu.SemaphoreType.DMA,
    ],
    compiler_params=pltpu.CompilerParams(use_tc_tiling_on_sc=False,
                                         needs_layout_passes=False,
                                         disable_bounds_checks=True),
)
def _gather_kernel(lab_hbm, w_hbm, out_hbm, lab_v, idxT_v, rows_v, outT_v,
                   sem_i, sem_g0, sem_g1, sem_o0, sem_o1):
    wid = lax.axis_index("s") * NC + lax.axis_index("c")
    sem_g = (sem_g0, sem_g1)
    sem_o = (sem_o0, sem_o1)
    iota = lax.iota(jnp.int32, 16)
    zeros = jnp.zeros((16,), jnp.int32)

    for c in range(NCHUNK):
        bt = wid * NCHUNK + c
        pltpu.async_copy(lab_hbm.at[pl.ds(bt * CB, CB)], lab_v, sem_i).wait()

        def lab_t(i, carry, c=c):
            h = i // (CB // 16)
            b0 = (i % (CB // 16)) * 16
            vals = plsc.load_gather(lab_v, [b0 + iota, zeros + h])
            idxT_v[c, h, pl.ds(b0, 16)] = vals
            return carry

        lax.fori_loop(0, HIST * (CB // 16), lab_t, 0)

    def fire_gathers(s, u):
        c = s // SPC
        h0 = (s % SPC) * G
        for hg in range(G):
            pltpu.async_copy(w_hbm.at[idxT_v.at[c, h0 + hg]],
                             rows_v.at[pl.ds((u * G + hg) * CB, CB)],
                             sem_g[u])

    def wait_gathers(s, u):
        c = s // SPC
        h0 = (s % SPC) * G
        for hg in range(G):
            pltpu.make_async_copy(w_hbm.at[idxT_v.at[c, h0 + hg]],
                                  rows_v.at[pl.ds((u * G + hg) * CB, CB)],
                                  sem_g[u]).wait()

    dcols = [zeros + d for d in range(DIM)]

    def transpose_rows(u):
        @plsc.parallel_loop(0, CB // 16, unroll=8)
        def tb(i):
            b0 = i * 16
            bidx = b0 + iota
            for hg in range(G):
                rvec = bidx + (u * G + hg) * CB
                for dt in range(DIM // 8):
                    for di in range(8):
                        vals = plsc.load_gather(rows_v,
                                                [rvec, dcols[dt * 8 + di]])
                        outT_v[u, hg, dt, di, pl.ds(b0, 16)] = vals

    def out_dst(s):
        c = s // SPC
        h0 = (s % SPC) * G
        bt = wid * NCHUNK + c
        return out_hbm.at[pl.ds(h0, G), :, bt]

    def fire_out(s, u):
        pltpu.async_copy(outT_v.at[u], out_dst(s), sem_o[u])

    def wait_out(s, u):
        pltpu.make_async_copy(outT_v.at[u], out_dst(s), sem_o[u]).wait()

    fire_gathers(0, 0)

    def body(p, carry):
        for u in (0, 1):
            v = 1 - u
            s = 2 * p + u
            wait_gathers(s, u)

            @pl.when(s + 1 < STEPS)
            def _():
                fire_gathers(s + 1, v)

            transpose_rows(u)
            fire_out(s, u)

            @pl.when(s >= 1)
            def _():
                wait_out(s - 1, v)
        return carry

    lax.fori_loop(0, N_PAIR, body, 0)
    wait_out(STEPS - 1, 1)


def kernel(labels, weight):
    o = _gather_kernel(labels.astype(jnp.int32), weight)
    return o.transpose(2, 4, 0, 1, 3).reshape(BATCH, HIST, DIM)
